# speedup vs baseline: 1.0293x; 1.0293x over previous
import jax
import jax.numpy as jnp
from jax.experimental import pallas as pl

N = 10000
HEADS = 4
H = 256


def _gcn_j(xx, W, b, src, dst, ns, nd):
    h = xx @ W
    h = h * ns[:, None]
    agg = jax.ops.segment_sum(h[src], dst, num_segments=N)
    return jax.nn.relu(agg * nd[:, None] + b)


def _gat_j(xx, W, al, ar, b, src, dst):
    z = (xx @ W).reshape(-1, HEADS, H)
    el = jnp.sum(z * al[None], axis=-1)
    er = jnp.sum(z * ar[None], axis=-1)
    e = jax.nn.leaky_relu(el[src] + er[dst], 0.2)
    ee = jnp.exp(e)
    s = jax.ops.segment_sum(ee, dst, num_segments=N)
    alpha = ee / (s[dst] + 1e-9)
    out = jax.ops.segment_sum(z[src] * alpha[:, :, None], dst, num_segments=N)
    return jax.nn.relu(out + b.reshape(HEADS, H)[None])


def _readout_j(h):
    return jnp.concatenate(
        [jnp.mean(h, axis=0, keepdims=True), jnp.max(h, axis=0, keepdims=True)], axis=1)


def kernel(x, edge_index, W1, b1, W2, b2, W3, b3, Wg1, al1, ar1, bg1, Wg2,
           al2, ar2, bg2, Wg3, al3, ar3, bg3, fc1_W, fc1_b, fc2_W, fc2_b):
    src = edge_index[0]
    dst = edge_index[1]
    ones = jnp.ones((src.shape[0],), jnp.float32)
    deg_out = jax.ops.segment_sum(ones, src, num_segments=N)
    deg_in = jax.ops.segment_sum(ones, dst, num_segments=N)
    ns = jnp.where(deg_out > 0, 1.0 / jnp.sqrt(jnp.maximum(deg_out, 1.0)), 0.0)
    nd = jnp.where(deg_in > 0, 1.0 / jnp.sqrt(jnp.maximum(deg_in, 1.0)), 0.0)

    gcn1 = _gcn_j(x, W1, b1, src, dst, ns, nd)
    spec1 = _readout_j(gcn1)
    gcn2 = _gcn_j(gcn1, W2, b2, src, dst, ns, nd)
    spec2 = _readout_j(gcn2)
    gcn3 = _gcn_j(gcn2, W3, b3, src, dst, ns, nd)
    spec3 = _readout_j(gcn3)
    gat1 = jnp.mean(_gat_j(x, Wg1, al1, ar1, bg1, src, dst), axis=1)
    gat2 = jnp.mean(_gat_j(gat1, Wg2, al2, ar2, bg2, src, dst), axis=1)
    spat2 = _readout_j(gat2)
    gat3 = jnp.mean(_gat_j(gat2, Wg3, al3, ar3, bg3, src, dst), axis=1)
    spat3 = jnp.concatenate(
        [jnp.mean(gat2, axis=0, keepdims=True), jnp.max(gat3, axis=0, keepdims=True)],
        axis=1)
    spec_merged = spec1 + spec2 + spec3
    spat_merged = spec1 + spat2 + spat3
    merged = jnp.concatenate([spec_merged, spat_merged], axis=1)
    fc1 = jax.nn.relu(merged @ fc1_W + fc1_b)
    fc2 = fc1 @ fc2_W + fc2_b
    return jax.nn.sigmoid(fc2)


# speedup vs baseline: 4.2223x; 4.1019x over previous
import functools
import jax
import jax.numpy as jnp
from jax import lax
from jax.experimental import pallas as pl
from jax.experimental.pallas import tpu as pltpu, tpu_sc as plsc

N = 10000
N_PAD = 10240
E = 320000
D_IN = 128
H = 256
HEADS = 4

K2 = 128
NCH2 = 20
NPC2 = N_PAD // NCH2
BUF2 = NPC2 + 128
ALIGN2 = 16 * K2
EPAD2 = E + NCH2 * ALIGN2

K8 = 64
NCH8 = 40
NPC8 = N_PAD // NCH8
BUF8 = NPC8 + 128
ALIGN8 = 16 * K8
EPAD8 = E + NCH8 * ALIGN8

_mesh = plsc.VectorSubcoreMesh(core_axis_name="c", subcore_axis_name="s")


def _build_layout(src_s, dst_s, nch, npc, k):
    align = 16 * k
    epad = E + nch * align
    cid = dst_s // npc
    cstart = jnp.searchsorted(dst_s, jnp.arange(nch, dtype=jnp.int32) * npc)
    cstart = cstart.astype(jnp.int32)
    cnt = jnp.diff(jnp.concatenate([cstart, jnp.array([E], jnp.int32)]))
    cntp = ((cnt + align - 1) // align) * align
    P = jnp.concatenate([jnp.zeros((1,), jnp.int32),
                         jnp.cumsum(cntp)[:-1].astype(jnp.int32)])
    pos = P[cid] + (jnp.arange(E, dtype=jnp.int32) - cstart[cid])
    srcp = jnp.zeros((epad,), jnp.int32).at[pos].set(src_s)
    dstl = jnp.full((epad,), npc, jnp.int32).at[pos].set(
        dst_s - cid.astype(jnp.int32) * npc)
    q = (cntp // 16).astype(jnp.int32)
    idx = jnp.arange(nch, dtype=jnp.int32) * 16
    bounds = (jnp.zeros((nch * 16,), jnp.int32)
              .at[idx].set(P).at[idx + 1].set(q))
    return srcp, dstl, bounds


def _sc_gcn(panels):
    scratch = [pltpu.VMEM((NCH2 * 16,), jnp.int32)]
    scratch += [pltpu.VMEM((K2,), jnp.int32)]
    scratch += [pltpu.VMEM((K2,), jnp.int32) for _ in range(panels)]
    scratch += [pltpu.VMEM((K2,), jnp.int32) for _ in range(panels)]
    scratch += [pltpu.VMEM((K2, 128), jnp.float32) for _ in range(panels)]
    scratch += [pltpu.VMEM((64, 128), jnp.float32)]
    scratch += [pltpu.VMEM_SHARED((BUF2, 128), jnp.float32)
                for _ in range(panels)]
    scratch += [pltpu.SemaphoreType.DMA for _ in range(panels)]

    @functools.partial(
        pl.kernel,
        out_type=jax.ShapeDtypeStruct((panels * NCH2 * NPC2, 128),
                                      jnp.float32),
        mesh=_mesh,
        scratch_types=scratch,
    )
    def k(featP, srcp, dstl, bounds, out, bounds_v, sidx2, *refs):
        pidx = refs[:panels]
        didx = refs[panels:2 * panels]
        rows = refs[2 * panels:3 * panels]
        zblk = refs[3 * panels]
        sbuf = refs[3 * panels + 1:3 * panels + 1 + panels]
        sems = refs[3 * panels + 1 + panels:]
        c = lax.axis_index("c")
        s = lax.axis_index("s")
        for i in range(64):
            for j in range(8):
                zblk[i, pl.ds(j * 16, 16)] = jnp.zeros((16,), jnp.float32)
        pltpu.sync_copy(bounds, bounds_v)

        def chunk_body(cc, chunk_carry):
            cg = c * (NCH2 // 2) + cc
            for p in range(panels):
                pltpu.sync_copy(
                    zblk.at[pl.ds(0, 40), :],
                    sbuf[p].at[pl.ds(pl.multiple_of(s * 40, 8), 40), :])
            plsc.subcore_barrier()
            bv = bounds_v[pl.ds(pl.multiple_of(cg * 16, 16), 16)]
            P = bv[0]
            q = bv[1]
            start = P + s * q
            nb = q // K2

            def body(i, carry):
                off = pl.multiple_of(start + i * K2, 8)
                pltpu.sync_copy(srcp.at[pl.ds(off, K2)], sidx2)
                for p in range(panels):
                    pltpu.sync_copy(dstl.at[pl.ds(off, K2)], didx[p])
                    if panels == 1:
                        pltpu.sync_copy(srcp.at[pl.ds(off, K2)], pidx[p])
                    else:
                        for j in range(K2 // 16):
                            sl = pl.ds(j * 16, 16)
                            pidx[p][sl] = sidx2[sl] * panels + p
                    pltpu.async_copy(featP.at[pidx[p]], rows[p], sems[p]).wait()
                    pltpu.sync_copy(rows[p], sbuf[p].at[didx[p]], add=True)
                return carry

            lax.fori_loop(0, nb, body, 0)
            plsc.subcore_barrier()
            for p in range(panels):
                pltpu.sync_copy(
                    sbuf[p].at[pl.ds(pl.multiple_of(s * 32, 8), 32), :],
                    out.at[pl.ds(pl.multiple_of(
                        (p * NCH2 + cg) * NPC2 + s * 32, 8), 32), :])
            plsc.subcore_barrier()
            return chunk_carry

        lax.fori_loop(0, NCH2 // 2, chunk_body, 0)

    return k


_sc_gcn1 = _sc_gcn(1)
_sc_gcn2 = _sc_gcn(2)


@functools.partial(
    pl.kernel,
    out_type=[jax.ShapeDtypeStruct((EPAD8, 16), jnp.float32),
              jax.ShapeDtypeStruct((NCH8 * BUF8, 16), jnp.float32)],
    mesh=_mesh,
    scratch_types=[
        pltpu.VMEM((NCH8 * 16,), jnp.int32),
        pltpu.VMEM((K8,), jnp.int32),
        pltpu.VMEM((K8,), jnp.int32),
        pltpu.VMEM((K8,), jnp.int32),
        pltpu.VMEM((K8,), jnp.int32),
        pltpu.VMEM((K8, 128), jnp.float32),
        pltpu.VMEM((K8, 128), jnp.float32),
        pltpu.VMEM((K8, 128), jnp.float32),
        pltpu.VMEM((K8, 16), jnp.float32),
        pltpu.VMEM((64, 128), jnp.float32),
        pltpu.VMEM((24, 128), jnp.float32),
        pltpu.VMEM((24, 16), jnp.float32),
        pltpu.VMEM_SHARED((BUF8, 128), jnp.float32),
        pltpu.SemaphoreType.DMA,
        pltpu.SemaphoreType.DMA,
    ],
)
def _sc_alpha(elT, erT, srcp, dstl, bounds, ee_out, s_out, bounds_v,
              sidx, didx, didx2, didxg, elb, erb, eeb, ee16, zblk,
              sstage, s16b, s2, sem, sem2):
    c = lax.axis_index("c")
    s = lax.axis_index("s")
    for i in range(64):
        for j in range(8):
            zblk[i, pl.ds(j * 16, 16)] = jnp.zeros((16,), jnp.float32)
    pltpu.sync_copy(bounds, bounds_v)

    def chunk_body(cc, chunk_carry):
        cg = c * (NCH8 // 2) + cc
        base = cg * NPC8
        pltpu.sync_copy(zblk.at[pl.ds(0, 24), :],
                        s2.at[pl.ds(pl.multiple_of(s * 24, 8), 24), :])
        plsc.subcore_barrier()
        bv = bounds_v[pl.ds(pl.multiple_of(cg * 16, 16), 16)]
        P = bv[0]
        q = bv[1]
        start = P + s * q
        nb = q // K8

        def body(i, carry):
            off = pl.multiple_of(start + i * K8, 8)
            pltpu.sync_copy(srcp.at[pl.ds(off, K8)], sidx)
            pltpu.sync_copy(dstl.at[pl.ds(off, K8)], didx)
            pltpu.sync_copy(dstl.at[pl.ds(off, K8)], didx2)
            for j in range(K8 // 16):
                sl = pl.ds(j * 16, 16)
                didxg[sl] = didx2[sl] + base
            pltpu.async_copy(elT.at[sidx], elb, sem).wait()
            pltpu.async_copy(erT.at[didxg], erb, sem).wait()
            for e in range(K8):
                v = elb[e, pl.ds(0, 16)] + erb[e, pl.ds(0, 16)]
                v = jnp.where(v > 0, v, 0.2 * v)
                v = jnp.exp(v)
                v = jnp.where(lax.iota(jnp.int32, 16) < 4, v, 0.0)
                eeb[e, pl.ds(0, 16)] = v
                ee16[e, pl.ds(0, 16)] = v
                for j in range(1, 8):
                    eeb[e, pl.ds(j * 16, 16)] = jnp.zeros((16,), jnp.float32)
            pltpu.sync_copy(eeb, s2.at[didx], add=True)
            pltpu.sync_copy(ee16, ee_out.at[pl.ds(off, K8), :])
            return carry

        lax.fori_loop(0, nb, body, 0)
        plsc.subcore_barrier()
        pltpu.sync_copy(s2.at[pl.ds(pl.multiple_of(s * 24, 8), 24), :], sstage)
        for r in range(24):
            s16b[r, pl.ds(0, 16)] = sstage[r, pl.ds(0, 16)]
        pltpu.sync_copy(
            s16b,
            s_out.at[pl.ds(pl.multiple_of(cg * BUF8 + s * 24, 8), 24), :])
        plsc.subcore_barrier()
        return chunk_carry

    lax.fori_loop(0, NCH8 // 2, chunk_body, 0)
    return


_NP8 = 8

_gat_scratch = [pltpu.VMEM((NCH8 * 16,), jnp.int32)]
_gat_scratch += [pltpu.VMEM((K8,), jnp.int32)]
_gat_scratch += [pltpu.VMEM((K8,), jnp.int32)]
_gat_scratch += [pltpu.VMEM((K8,), jnp.int32) for _ in range(_NP8)]
_gat_scratch += [pltpu.VMEM((K8,), jnp.int32) for _ in range(_NP8)]
_gat_scratch += [pltpu.VMEM((K8, 128), jnp.float32) for _ in range(_NP8)]
_gat_scratch += [pltpu.VMEM((K8 * 16,), jnp.float32)]
_gat_scratch += [pltpu.VMEM((BUF8 * 16,), jnp.float32)]
_gat_scratch += [pltpu.VMEM((64, 128), jnp.float32)]
_gat_scratch += [pltpu.VMEM_SHARED((BUF8, 128), jnp.float32)
                 for _ in range(_NP8)]
_gat_scratch += [pltpu.SemaphoreType.DMA for _ in range(_NP8)]


@functools.partial(
    pl.kernel,
    out_type=jax.ShapeDtypeStruct((_NP8 * NCH8 * NPC8, 128), jnp.float32),
    mesh=_mesh,
    scratch_types=_gat_scratch,
)
def _sc_gat(z8, srcp, dstl, ee_flat, s_flat, bounds, out, bounds_v,
            sidx2, didx2, *refs):
    pidx = refs[:_NP8]
    didx = refs[_NP8:2 * _NP8]
    zp = refs[2 * _NP8:3 * _NP8]
    eebuf = refs[3 * _NP8]
    svbuf = refs[3 * _NP8 + 1]
    zblk = refs[3 * _NP8 + 2]
    sbuf = refs[3 * _NP8 + 3:3 * _NP8 + 3 + _NP8]
    sems = refs[3 * _NP8 + 3 + _NP8:]
    c = lax.axis_index("c")
    s = lax.axis_index("s")
    for i in range(64):
        for j in range(8):
            zblk[i, pl.ds(j * 16, 16)] = jnp.zeros((16,), jnp.float32)
    pltpu.sync_copy(bounds, bounds_v)

    def chunk_body(cc, chunk_carry):
        cg = c * (NCH8 // 2) + cc
        for p in range(_NP8):
            pltpu.sync_copy(zblk.at[pl.ds(0, 24), :],
                            sbuf[p].at[pl.ds(pl.multiple_of(s * 24, 8), 24), :])
        pltpu.sync_copy(
            s_flat.at[pl.ds(pl.multiple_of(cg * BUF8 * 16, 8), BUF8 * 16)],
            svbuf)
        plsc.subcore_barrier()
        bv = bounds_v[pl.ds(pl.multiple_of(cg * 16, 16), 16)]
        P = bv[0]
        q = bv[1]
        start = P + s * q
        nb = q // K8

        def body(i, carry):
            off = pl.multiple_of(start + i * K8, 8)
            pltpu.sync_copy(srcp.at[pl.ds(off, K8)], sidx2)
            pltpu.sync_copy(dstl.at[pl.ds(off, K8)], didx2)
            pltpu.sync_copy(
                ee_flat.at[pl.ds(pl.multiple_of(off * 16, 8), K8 * 16)],
                eebuf)
            for p in range(_NP8):
                pltpu.sync_copy(dstl.at[pl.ds(off, K8)], didx[p])
                for j in range(K8 // 16):
                    sl = pl.ds(j * 16, 16)
                    pidx[p][sl] = sidx2[sl] * _NP8 + p
                pltpu.async_copy(z8.at[pidx[p]], zp[p], sems[p]).wait()

            def gbody(g, carry2):
                dv = didx2[pl.ds(pl.multiple_of(g * 16, 16), 16)]
                for t in range(16):
                    e = g * 16 + t
                    ev = eebuf[pl.ds(pl.multiple_of(e * 16, 16), 16)]
                    d = dv[t]
                    sv = svbuf[pl.ds(pl.multiple_of(d * 16, 16), 16)]
                    av = ev / (sv + 1e-9)
                    a = (av[0], av[1], av[2], av[3])
                    for p in range(_NP8):
                        bb = jnp.full((16,), a[p // 2], jnp.float32)
                        for j in range(8):
                            sl = pl.ds(j * 16, 16)
                            zp[p][e, sl] = zp[p][e, sl] * bb
                return carry2

            lax.fori_loop(0, K8 // 16, gbody, 0)
            for p in range(_NP8):
                pltpu.sync_copy(zp[p], sbuf[p].at[didx[p]], add=True)
            return carry

        lax.fori_loop(0, nb, body, 0)
        plsc.subcore_barrier()
        for p in range(_NP8):
            pltpu.sync_copy(
                sbuf[p].at[pl.ds(pl.multiple_of(s * 16, 8), 16), :],
                out.at[pl.ds(pl.multiple_of(
                    (p * NCH8 + cg) * NPC8 + s * 16, 8), 16), :])
        plsc.subcore_barrier()
        return chunk_carry

    lax.fori_loop(0, NCH8 // 2, chunk_body, 0)
    return


_BR = 1024
_GRID = N_PAD // _BR


def _norm_from_deg(dv):
    return jnp.where(dv > 0, lax.rsqrt(jnp.maximum(dv, 1.0)), 0.0)


def _tc_scale(a, deg_r):
    f = a.shape[1]

    def body(a_ref, d_ref, o_ref):
        ns = _norm_from_deg(d_ref[...])
        if f > 128:
            ns = jnp.concatenate([ns] * (f // 128), axis=1)
        o_ref[...] = a_ref[...] * ns

    return pl.pallas_call(
        body,
        grid=(_GRID,),
        in_specs=[pl.BlockSpec((_BR, f), lambda i: (i, 0)),
                  pl.BlockSpec((_BR, 128), lambda i: (i, 0))],
        out_specs=pl.BlockSpec((_BR, f), lambda i: (i, 0)),
        out_shape=jax.ShapeDtypeStruct((N_PAD, f), jnp.float32),
    )(a, deg_r)


def _tc_mm(a, w, deg_r, b):
    kd, m = w.shape

    def body(a_ref, w_ref, d_ref, b_ref, o_ref):
        acc = jnp.dot(a_ref[...], w_ref[...],
                      preferred_element_type=jnp.float32)
        nd = _norm_from_deg(d_ref[...])
        if m > 128:
            nd = jnp.concatenate([nd] * (m // 128), axis=1)
        o_ref[...] = jnp.maximum(acc * nd + b_ref[...], 0.0)

    return pl.pallas_call(
        body,
        grid=(_GRID,),
        in_specs=[pl.BlockSpec((_BR, kd), lambda i: (i, 0)),
                  pl.BlockSpec((kd, m), lambda i: (0, 0)),
                  pl.BlockSpec((_BR, 128), lambda i: (i, 0)),
                  pl.BlockSpec((1, m), lambda i: (0, 0))],
        out_specs=pl.BlockSpec((_BR, m), lambda i: (i, 0)),
        out_shape=jax.ShapeDtypeStruct((N_PAD, m), jnp.float32),
    )(a, w, deg_r, b)


def _tc_mm_z(a, w, alp, arp):
    kd, m = w.shape

    def body(a_ref, w_ref, al_ref, ar_ref, z_ref, el_ref, er_ref):
        z = jnp.dot(a_ref[...], w_ref[...],
                    preferred_element_type=jnp.float32)
        z_ref[...] = z
        z3 = z.reshape(_BR, HEADS, H)
        al = al_ref[...][0:HEADS]
        ar = ar_ref[...][0:HEADS]
        el = jnp.sum(z3 * al[None], axis=-1)
        er = jnp.sum(z3 * ar[None], axis=-1)
        el_ref[...] = jnp.pad(el, ((0, 0), (0, 128 - HEADS)))
        er_ref[...] = jnp.pad(er, ((0, 0), (0, 128 - HEADS)))

    return pl.pallas_call(
        body,
        grid=(_GRID,),
        in_specs=[pl.BlockSpec((_BR, kd), lambda i: (i, 0)),
                  pl.BlockSpec((kd, m), lambda i: (0, 0)),
                  pl.BlockSpec((8, H), lambda i: (0, 0)),
                  pl.BlockSpec((8, H), lambda i: (0, 0))],
        out_specs=[pl.BlockSpec((_BR, m), lambda i: (i, 0)),
                   pl.BlockSpec((_BR, 128), lambda i: (i, 0)),
                   pl.BlockSpec((_BR, 128), lambda i: (i, 0))],
        out_shape=[jax.ShapeDtypeStruct((N_PAD, m), jnp.float32),
                   jax.ShapeDtypeStruct((N_PAD, 128), jnp.float32),
                   jax.ShapeDtypeStruct((N_PAD, 128), jnp.float32)],
    )(a, w, alp, arp)


def _tc_gatpost(agg, bg):
    def body(a_ref, b_ref, o_ref):
        v = jnp.maximum(a_ref[...] + b_ref[...], 0.0)
        o_ref[...] = jnp.mean(v.reshape(_BR, HEADS, H), axis=1)

    return pl.pallas_call(
        body,
        grid=(_GRID,),
        in_specs=[pl.BlockSpec((_BR, HEADS * H), lambda i: (i, 0)),
                  pl.BlockSpec((1, HEADS * H), lambda i: (0, 0))],
        out_specs=pl.BlockSpec((_BR, H), lambda i: (i, 0)),
        out_shape=jax.ShapeDtypeStruct((N_PAD, H), jnp.float32),
    )(agg, bg)


def _tc_pool(h):
    def body(h_ref, o_ref):
        i = pl.program_id(0)
        rid = i * _BR + lax.broadcasted_iota(jnp.int32, (_BR, 1), 0)
        msk = rid < N
        v = h_ref[...]
        vs = jnp.where(msk, v, 0.0)
        vm = jnp.where(msk, v, -jnp.inf)
        psum = jnp.sum(vs, axis=0, keepdims=True)
        pmax = jnp.max(vm, axis=0, keepdims=True)

        @pl.when(i == 0)
        def _():
            o_ref[...] = jnp.zeros_like(o_ref)

        o_ref[0:1, :] += psum
        o_ref[1:2, :] = jnp.maximum(o_ref[1:2, :], pmax)

        @pl.when(i == _GRID - 1)
        def _():
            o_ref[0:1, :] = o_ref[0:1, :] * (1.0 / N)

    return pl.pallas_call(
        body,
        grid=(_GRID,),
        in_specs=[pl.BlockSpec((_BR, H), lambda i: (i, 0))],
        out_specs=pl.BlockSpec((8, H), lambda i: (0, 0)),
        out_shape=jax.ShapeDtypeStruct((8, H), jnp.float32),
    )(h)


def _tc_head(p1, p2, p3, pg2, pg3, w1, b1, w2, b2):
    def body(p1r, p2r, p3r, g2r, g3r, w1r, b1r, w2r, b2r, o_ref):
        sm = p1r[0:1] + p2r[0:1] + p3r[0:1]
        sx = p1r[1:2] + p2r[1:2] + p3r[1:2]
        pm = p1r[0:1] + 2.0 * g2r[0:1]
        px = p1r[1:2] + g2r[1:2] + g3r[1:2]
        merged = jnp.concatenate([sm, sx, pm, px], axis=1)
        f1 = jnp.maximum(
            jnp.dot(merged, w1r[...], preferred_element_type=jnp.float32)
            + b1r[...], 0.0)
        f2 = (jnp.dot(f1, w2r[...], preferred_element_type=jnp.float32)
              + b2r[...])
        o_ref[...] = jnp.broadcast_to(jax.nn.sigmoid(f2), (8, 128))

    specs = [pl.BlockSpec(p.shape, lambda i: (0, 0))
             for p in (p1, p2, p3, pg2, pg3, w1, b1, w2, b2)]
    return pl.pallas_call(
        body,
        grid=(1,),
        in_specs=specs,
        out_specs=pl.BlockSpec((8, 128), lambda i: (0, 0)),
        out_shape=jax.ShapeDtypeStruct((8, 128), jnp.float32),
    )(p1, p2, p3, pg2, pg3, w1, b1, w2, b2)


def _unpanel(aggP, panels, nch, npc):
    a = aggP.reshape(panels, nch, npc, 128)
    return jnp.moveaxis(a, 0, 2).reshape(nch * npc, panels * 128)


def kernel(x, edge_index, W1, b1, W2, b2, W3, b3, Wg1, al1, ar1, bg1, Wg2,
           al2, ar2, bg2, Wg3, al3, ar3, bg3, fc1_W, fc1_b, fc2_W, fc2_b):
    src = edge_index[0]
    dst = edge_index[1]
    order = jnp.argsort(dst)
    dst_s = dst[order]
    src_s = src[order]
    srcp2, dstl2, bounds2 = _build_layout(src_s, dst_s, NCH2, NPC2, K2)
    srcp8, dstl8, bounds8 = _build_layout(src_s, dst_s, NCH8, NPC8, K8)
    arange_n = jnp.arange(N, dtype=jnp.int32)
    lo = jnp.searchsorted(dst_s, arange_n)
    hi = jnp.searchsorted(dst_s, arange_n + 1)
    deg_in = (hi - lo).astype(jnp.float32)
    srt = jnp.sort(src)
    lo2 = jnp.searchsorted(srt, arange_n)
    hi2 = jnp.searchsorted(srt, arange_n + 1)
    deg_out = (hi2 - lo2).astype(jnp.float32)
    degout_r = jnp.broadcast_to(
        jnp.pad(deg_out, (0, N_PAD - N))[:, None], (N_PAD, 128))
    degin_r = jnp.broadcast_to(
        jnp.pad(deg_in, (0, N_PAD - N))[:, None], (N_PAD, 128))

    x_pad = jnp.pad(x, ((0, N_PAD - N), (0, 0)))

    def gcn_layer(hin, W, b, panels, segk):
        t = _tc_scale(hin, degout_r)
        tP = t.reshape(N_PAD * panels, 128)
        aggP = segk(tP, srcp2, dstl2, bounds2)
        agg = _unpanel(aggP, panels, NCH2, NPC2)
        return _tc_mm(agg, W, degin_r, b.reshape(1, -1))

    gcn1 = gcn_layer(x_pad, W1, b1, 1, _sc_gcn1)
    gcn2 = gcn_layer(gcn1, W2, b2, 2, _sc_gcn2)
    gcn3 = gcn_layer(gcn2, W3, b3, 2, _sc_gcn2)

    def gat_layer(hin, Wg, al, ar, bg):
        alp = jnp.pad(al, ((0, 8 - HEADS), (0, 0)))
        arp = jnp.pad(ar, ((0, 8 - HEADS), (0, 0)))
        z, elp, erp = _tc_mm_z(hin, Wg, alp, arp)
        elp8 = jnp.pad(elp, ((0, 8), (0, 0)))
        erp8 = jnp.pad(erp, ((0, 8), (0, 0)))
        ee, s16 = _sc_alpha(elp8, erp8, srcp8, dstl8, bounds8)
        z8 = z.reshape(N_PAD * _NP8, 128)
        aggP = _sc_gat(z8, srcp8, dstl8, ee.reshape(-1), s16.reshape(-1),
                       bounds8)
        agg = _unpanel(aggP, _NP8, NCH8, NPC8)
        return _tc_gatpost(agg, bg.reshape(1, -1))

    gat1 = gat_layer(x_pad, Wg1, al1, ar1, bg1)
    gat2 = gat_layer(gat1, Wg2, al2, ar2, bg2)
    gat3 = gat_layer(gat2, Wg3, al3, ar3, bg3)

    p1 = _tc_pool(gcn1)
    p2 = _tc_pool(gcn2)
    p3 = _tc_pool(gcn3)
    pg2 = _tc_pool(gat2)
    pg3 = _tc_pool(gat3)
    w2p = jnp.pad(fc2_W, ((0, 0), (0, 128 - fc2_W.shape[1])))
    b2p = jnp.pad(fc2_b, (0, 128 - fc2_b.shape[0])).reshape(1, 128)
    out = _tc_head(p1, p2, p3, pg2, pg3, fc1_W, fc1_b.reshape(1, -1),
                   w2p, b2p)
    return out[0:1, 0:2]


# speedup vs baseline: 4.7522x; 1.1255x over previous
import functools
import jax
import jax.numpy as jnp
from jax import lax
from jax.experimental import pallas as pl
from jax.experimental.pallas import tpu as pltpu, tpu_sc as plsc

N = 10000
N_PAD = 10240
E = 320000
D_IN = 128
H = 256
HEADS = 4

K2 = 128
NCH2 = 20
NPC2 = N_PAD // NCH2
BUF2 = NPC2 + 128
ALIGN2 = 16 * K2
EPAD2 = E + NCH2 * ALIGN2

K8 = 64
NCH8 = 40
NPC8 = N_PAD // NCH8
BUF8 = NPC8 + 128
ALIGN8 = 16 * K8
EPAD8 = E + NCH8 * ALIGN8

_mesh = plsc.VectorSubcoreMesh(core_axis_name="c", subcore_axis_name="s")


def _build_layout(src_s, dst_s, nch, npc, k):
    align = 16 * k
    epad = E + nch * align
    cid = dst_s // npc
    cstart = jnp.searchsorted(dst_s, jnp.arange(nch, dtype=jnp.int32) * npc)
    cstart = cstart.astype(jnp.int32)
    cnt = jnp.diff(jnp.concatenate([cstart, jnp.array([E], jnp.int32)]))
    cntp = ((cnt + align - 1) // align) * align
    P = jnp.concatenate([jnp.zeros((1,), jnp.int32),
                         jnp.cumsum(cntp)[:-1].astype(jnp.int32)])
    pos = P[cid] + (jnp.arange(E, dtype=jnp.int32) - cstart[cid])
    srcp = jnp.zeros((epad,), jnp.int32).at[pos].set(src_s)
    dstl = jnp.full((epad,), npc, jnp.int32).at[pos].set(
        dst_s - cid.astype(jnp.int32) * npc)
    q = (cntp // 16).astype(jnp.int32)
    idx = jnp.arange(nch, dtype=jnp.int32) * 16
    bounds = (jnp.zeros((nch * 16,), jnp.int32)
              .at[idx].set(P).at[idx + 1].set(q))
    return srcp, dstl, bounds


def _sc_gcn(panels):
    scratch = [pltpu.VMEM((NCH2 * 16,), jnp.int32)]
    scratch += [pltpu.VMEM((K2,), jnp.int32)]
    scratch += [pltpu.VMEM((K2,), jnp.int32) for _ in range(panels)]
    scratch += [pltpu.VMEM((K2,), jnp.int32) for _ in range(panels)]
    scratch += [pltpu.VMEM((K2, 128), jnp.float32) for _ in range(panels)]
    scratch += [pltpu.VMEM((64, 128), jnp.float32)]
    scratch += [pltpu.VMEM_SHARED((BUF2, 128), jnp.float32)
                for _ in range(panels)]
    scratch += [pltpu.SemaphoreType.DMA for _ in range(panels)]

    @functools.partial(
        pl.kernel,
        out_type=jax.ShapeDtypeStruct((panels * NCH2 * NPC2, 128),
                                      jnp.float32),
        mesh=_mesh,
        scratch_types=scratch,
    )
    def k(featP, srcp, dstl, bounds, out, bounds_v, sidx2, *refs):
        pidx = refs[:panels]
        didx = refs[panels:2 * panels]
        rows = refs[2 * panels:3 * panels]
        zblk = refs[3 * panels]
        sbuf = refs[3 * panels + 1:3 * panels + 1 + panels]
        sems = refs[3 * panels + 1 + panels:]
        c = lax.axis_index("c")
        s = lax.axis_index("s")
        for i in range(64):
            for j in range(8):
                zblk[i, pl.ds(j * 16, 16)] = jnp.zeros((16,), jnp.float32)
        pltpu.sync_copy(bounds, bounds_v)

        def chunk_body(cc, chunk_carry):
            cg = c * (NCH2 // 2) + cc
            for p in range(panels):
                pltpu.sync_copy(
                    zblk.at[pl.ds(0, 40), :],
                    sbuf[p].at[pl.ds(pl.multiple_of(s * 40, 8), 40), :])
            plsc.subcore_barrier()
            bv = bounds_v[pl.ds(pl.multiple_of(cg * 16, 16), 16)]
            P = bv[0]
            q = bv[1]
            start = P + s * q
            nb = q // K2

            def body(i, carry):
                off = pl.multiple_of(start + i * K2, 8)
                pltpu.sync_copy(srcp.at[pl.ds(off, K2)], sidx2)
                for p in range(panels):
                    pltpu.sync_copy(dstl.at[pl.ds(off, K2)], didx[p])
                    if panels == 1:
                        pltpu.sync_copy(srcp.at[pl.ds(off, K2)], pidx[p])
                    else:
                        for j in range(K2 // 16):
                            sl = pl.ds(j * 16, 16)
                            pidx[p][sl] = sidx2[sl] * panels + p
                    pltpu.async_copy(featP.at[pidx[p]], rows[p], sems[p]).wait()
                    pltpu.sync_copy(rows[p], sbuf[p].at[didx[p]], add=True)
                return carry

            lax.fori_loop(0, nb, body, 0)
            plsc.subcore_barrier()
            for p in range(panels):
                pltpu.sync_copy(
                    sbuf[p].at[pl.ds(pl.multiple_of(s * 32, 8), 32), :],
                    out.at[pl.ds(pl.multiple_of(
                        (p * NCH2 + cg) * NPC2 + s * 32, 8), 32), :])
            plsc.subcore_barrier()
            return chunk_carry

        lax.fori_loop(0, NCH2 // 2, chunk_body, 0)

    return k


_sc_gcn1 = _sc_gcn(1)
_sc_gcn2 = _sc_gcn(2)


@functools.partial(
    pl.kernel,
    out_type=[jax.ShapeDtypeStruct((EPAD8, 16), jnp.float32),
              jax.ShapeDtypeStruct((NCH8 * BUF8, 16), jnp.float32)],
    mesh=_mesh,
    scratch_types=[
        pltpu.VMEM((NCH8 * 16,), jnp.int32),
        pltpu.VMEM((K8,), jnp.int32),
        pltpu.VMEM((K8,), jnp.int32),
        pltpu.VMEM((K8,), jnp.int32),
        pltpu.VMEM((K8,), jnp.int32),
        pltpu.VMEM((K8, 128), jnp.float32),
        pltpu.VMEM((K8, 128), jnp.float32),
        pltpu.VMEM((K8, 128), jnp.float32),
        pltpu.VMEM((K8, 16), jnp.float32),
        pltpu.VMEM((64, 128), jnp.float32),
        pltpu.VMEM((24, 128), jnp.float32),
        pltpu.VMEM((24, 16), jnp.float32),
        pltpu.VMEM_SHARED((BUF8, 128), jnp.float32),
        pltpu.SemaphoreType.DMA,
        pltpu.SemaphoreType.DMA,
    ],
)
def _sc_alpha(elT, erT, srcp, dstl, bounds, ee_out, s_out, bounds_v,
              sidx, didx, didx2, didxg, elb, erb, eeb, ee16, zblk,
              sstage, s16b, s2, sem, sem2):
    c = lax.axis_index("c")
    s = lax.axis_index("s")
    for i in range(64):
        for j in range(8):
            zblk[i, pl.ds(j * 16, 16)] = jnp.zeros((16,), jnp.float32)
    pltpu.sync_copy(bounds, bounds_v)

    def chunk_body(cc, chunk_carry):
        cg = c * (NCH8 // 2) + cc
        base = cg * NPC8
        pltpu.sync_copy(zblk.at[pl.ds(0, 24), :],
                        s2.at[pl.ds(pl.multiple_of(s * 24, 8), 24), :])
        plsc.subcore_barrier()
        bv = bounds_v[pl.ds(pl.multiple_of(cg * 16, 16), 16)]
        P = bv[0]
        q = bv[1]
        start = P + s * q
        nb = q // K8

        def body(i, carry):
            off = pl.multiple_of(start + i * K8, 8)
            pltpu.sync_copy(srcp.at[pl.ds(off, K8)], sidx)
            pltpu.sync_copy(dstl.at[pl.ds(off, K8)], didx)
            pltpu.sync_copy(dstl.at[pl.ds(off, K8)], didx2)
            for j in range(K8 // 16):
                sl = pl.ds(j * 16, 16)
                didxg[sl] = didx2[sl] + base
            pltpu.async_copy(elT.at[sidx], elb, sem).wait()
            pltpu.async_copy(erT.at[didxg], erb, sem).wait()
            for e in range(K8):
                v = elb[e, pl.ds(0, 16)] + erb[e, pl.ds(0, 16)]
                v = jnp.where(v > 0, v, 0.2 * v)
                v = jnp.exp(v)
                v = jnp.where(lax.iota(jnp.int32, 16) < 4, v, 0.0)
                eeb[e, pl.ds(0, 16)] = v
                ee16[e, pl.ds(0, 16)] = v
                for j in range(1, 8):
                    eeb[e, pl.ds(j * 16, 16)] = jnp.zeros((16,), jnp.float32)
            pltpu.sync_copy(eeb, s2.at[didx], add=True)
            pltpu.sync_copy(ee16, ee_out.at[pl.ds(off, K8), :])
            return carry

        lax.fori_loop(0, nb, body, 0)
        plsc.subcore_barrier()
        pltpu.sync_copy(s2.at[pl.ds(pl.multiple_of(s * 24, 8), 24), :], sstage)
        for r in range(24):
            s16b[r, pl.ds(0, 16)] = sstage[r, pl.ds(0, 16)]
        pltpu.sync_copy(
            s16b,
            s_out.at[pl.ds(pl.multiple_of(cg * BUF8 + s * 24, 8), 24), :])
        plsc.subcore_barrier()
        return chunk_carry

    lax.fori_loop(0, NCH8 // 2, chunk_body, 0)
    return


_NP8 = 8

_NPAIR = 4

_gat_scratch = [pltpu.VMEM((NCH8 * 16,), jnp.int32)]
_gat_scratch += [pltpu.VMEM((K8,), jnp.int32)]
_gat_scratch += [pltpu.VMEM((K8,), jnp.int32)]
_gat_scratch += [pltpu.VMEM((2 * K8,), jnp.int32) for _ in range(_NPAIR)]
_gat_scratch += [pltpu.VMEM((2 * K8,), jnp.int32) for _ in range(_NPAIR)]
_gat_scratch += [pltpu.VMEM((2 * K8, 128), jnp.float32) for _ in range(_NPAIR)]
_gat_scratch += [pltpu.VMEM((K8 * 16,), jnp.float32)]
_gat_scratch += [pltpu.VMEM((BUF8 * 16,), jnp.float32)]
_gat_scratch += [pltpu.VMEM((64, 128), jnp.float32)]
_gat_scratch += [pltpu.VMEM_SHARED((_NP8 * BUF8, 128), jnp.float32)]
_gat_scratch += [pltpu.SemaphoreType.DMA for _ in range(_NPAIR)]


@functools.partial(
    pl.kernel,
    out_type=jax.ShapeDtypeStruct((_NP8 * NCH8 * NPC8, 128), jnp.float32),
    mesh=_mesh,
    scratch_types=_gat_scratch,
)
def _sc_gat(z8, srcp, dstl, ee_flat, s_flat, bounds, out, bounds_v,
            sidx2, didx2, *refs):
    pidx = refs[:_NPAIR]
    didxs = refs[_NPAIR:2 * _NPAIR]
    zp = refs[2 * _NPAIR:3 * _NPAIR]
    eebuf = refs[3 * _NPAIR]
    svbuf = refs[3 * _NPAIR + 1]
    zblk = refs[3 * _NPAIR + 2]
    sbuf = refs[3 * _NPAIR + 3]
    sems = refs[3 * _NPAIR + 4:]
    c = lax.axis_index("c")
    s = lax.axis_index("s")
    for i in range(64):
        for j in range(8):
            zblk[i, pl.ds(j * 16, 16)] = jnp.zeros((16,), jnp.float32)
    pltpu.sync_copy(bounds, bounds_v)

    def chunk_body(cc, chunk_carry):
        cg = c * (NCH8 // 2) + cc
        for kk in range(3):
            pltpu.sync_copy(
                zblk,
                sbuf.at[pl.ds(pl.multiple_of(s * 192 + kk * 64, 8), 64), :])
        pltpu.sync_copy(
            s_flat.at[pl.ds(pl.multiple_of(cg * BUF8 * 16, 8), BUF8 * 16)],
            svbuf)
        plsc.subcore_barrier()
        bv = bounds_v[pl.ds(pl.multiple_of(cg * 16, 16), 16)]
        P = bv[0]
        q = bv[1]
        start = P + s * q
        nb = q // K8

        def body(i, carry):
            off = pl.multiple_of(start + i * K8, 8)
            pltpu.sync_copy(srcp.at[pl.ds(off, K8)], sidx2)
            pltpu.sync_copy(dstl.at[pl.ds(off, K8)], didx2)
            pltpu.sync_copy(
                ee_flat.at[pl.ds(pl.multiple_of(off * 16, 8), K8 * 16)],
                eebuf)
            for pp in range(_NPAIR):
                for j in range(K8 // 16):
                    sl = pl.ds(j * 16, 16)
                    sl2 = pl.ds(K8 + j * 16, 16)
                    s_v = sidx2[sl]
                    d_v = didx2[sl]
                    pidx[pp][sl] = s_v * _NP8 + 2 * pp
                    pidx[pp][sl2] = s_v * _NP8 + 2 * pp + 1
                    didxs[pp][sl] = d_v + (2 * pp) * BUF8
                    didxs[pp][sl2] = d_v + (2 * pp + 1) * BUF8
                pltpu.async_copy(z8.at[pidx[pp]], zp[pp], sems[pp]).wait()

            def gbody(g, carry2):
                for t in range(16):
                    e = g * 16 + t
                    ev = eebuf[pl.ds(pl.multiple_of(e * 16, 16), 16)]
                    dv = didx2[pl.ds(pl.multiple_of(g * 16, 16), 16)]
                    d = dv[t]
                    sv = svbuf[pl.ds(pl.multiple_of(d * 16, 16), 16)]
                    av = ev / (sv + 1e-9)
                    a = (av[0], av[1], av[2], av[3])
                    for pp in range(_NPAIR):
                        bb = jnp.full((16,), a[pp], jnp.float32)
                        for j in range(8):
                            sl = pl.ds(j * 16, 16)
                            zp[pp][e, sl] = zp[pp][e, sl] * bb
                            zp[pp][K8 + e, sl] = zp[pp][K8 + e, sl] * bb
                return carry2

            lax.fori_loop(0, K8 // 16, gbody, 0)
            for pp in range(_NPAIR):
                pltpu.sync_copy(zp[pp], sbuf.at[didxs[pp]], add=True)
            return carry

        lax.fori_loop(0, nb, body, 0)
        plsc.subcore_barrier()
        for p in range(_NP8):
            pltpu.sync_copy(
                sbuf.at[pl.ds(pl.multiple_of(p * BUF8 + s * 16, 8), 16), :],
                out.at[pl.ds(pl.multiple_of(
                    (p * NCH8 + cg) * NPC8 + s * 16, 8), 16), :])
        plsc.subcore_barrier()
        return chunk_carry

    lax.fori_loop(0, NCH8 // 2, chunk_body, 0)
    return


_BR = 1024
_GRID = N_PAD // _BR


def _norm_from_deg(dv):
    return jnp.where(dv > 0, lax.rsqrt(jnp.maximum(dv, 1.0)), 0.0)


def _tc_scale(a, deg_r):
    f = a.shape[1]

    def body(a_ref, d_ref, o_ref):
        ns = _norm_from_deg(d_ref[...])
        if f > 128:
            ns = jnp.concatenate([ns] * (f // 128), axis=1)
        o_ref[...] = a_ref[...] * ns

    return pl.pallas_call(
        body,
        grid=(_GRID,),
        in_specs=[pl.BlockSpec((_BR, f), lambda i: (i, 0)),
                  pl.BlockSpec((_BR, 128), lambda i: (i, 0))],
        out_specs=pl.BlockSpec((_BR, f), lambda i: (i, 0)),
        out_shape=jax.ShapeDtypeStruct((N_PAD, f), jnp.float32),
    )(a, deg_r)


def _tc_mm(a, w, deg_r, b):
    kd, m = w.shape

    def body(a_ref, w_ref, d_ref, b_ref, o_ref):
        acc = jnp.dot(a_ref[...], w_ref[...],
                      preferred_element_type=jnp.float32)
        nd = _norm_from_deg(d_ref[...])
        if m > 128:
            nd = jnp.concatenate([nd] * (m // 128), axis=1)
        o_ref[...] = jnp.maximum(acc * nd + b_ref[...], 0.0)

    return pl.pallas_call(
        body,
        grid=(_GRID,),
        in_specs=[pl.BlockSpec((_BR, kd), lambda i: (i, 0)),
                  pl.BlockSpec((kd, m), lambda i: (0, 0)),
                  pl.BlockSpec((_BR, 128), lambda i: (i, 0)),
                  pl.BlockSpec((1, m), lambda i: (0, 0))],
        out_specs=pl.BlockSpec((_BR, m), lambda i: (i, 0)),
        out_shape=jax.ShapeDtypeStruct((N_PAD, m), jnp.float32),
    )(a, w, deg_r, b)


def _tc_mm_z(a, w, alp, arp):
    kd, m = w.shape

    def body(a_ref, w_ref, al_ref, ar_ref, z_ref, el_ref, er_ref):
        z = jnp.dot(a_ref[...], w_ref[...],
                    preferred_element_type=jnp.float32)
        z_ref[...] = z
        z3 = z.reshape(_BR, HEADS, H)
        al = al_ref[...][0:HEADS]
        ar = ar_ref[...][0:HEADS]
        el = jnp.sum(z3 * al[None], axis=-1)
        er = jnp.sum(z3 * ar[None], axis=-1)
        el_ref[...] = jnp.pad(el, ((0, 0), (0, 128 - HEADS)))
        er_ref[...] = jnp.pad(er, ((0, 0), (0, 128 - HEADS)))

    return pl.pallas_call(
        body,
        grid=(_GRID,),
        in_specs=[pl.BlockSpec((_BR, kd), lambda i: (i, 0)),
                  pl.BlockSpec((kd, m), lambda i: (0, 0)),
                  pl.BlockSpec((8, H), lambda i: (0, 0)),
                  pl.BlockSpec((8, H), lambda i: (0, 0))],
        out_specs=[pl.BlockSpec((_BR, m), lambda i: (i, 0)),
                   pl.BlockSpec((_BR, 128), lambda i: (i, 0)),
                   pl.BlockSpec((_BR, 128), lambda i: (i, 0))],
        out_shape=[jax.ShapeDtypeStruct((N_PAD, m), jnp.float32),
                   jax.ShapeDtypeStruct((N_PAD, 128), jnp.float32),
                   jax.ShapeDtypeStruct((N_PAD, 128), jnp.float32)],
    )(a, w, alp, arp)


def _tc_gatpost(agg, bg):
    def body(a_ref, b_ref, o_ref):
        v = jnp.maximum(a_ref[...] + b_ref[...], 0.0)
        o_ref[...] = jnp.mean(v.reshape(_BR, HEADS, H), axis=1)

    return pl.pallas_call(
        body,
        grid=(_GRID,),
        in_specs=[pl.BlockSpec((_BR, HEADS * H), lambda i: (i, 0)),
                  pl.BlockSpec((1, HEADS * H), lambda i: (0, 0))],
        out_specs=pl.BlockSpec((_BR, H), lambda i: (i, 0)),
        out_shape=jax.ShapeDtypeStruct((N_PAD, H), jnp.float32),
    )(agg, bg)


def _tc_pool(h):
    def body(h_ref, o_ref):
        i = pl.program_id(0)
        rid = i * _BR + lax.broadcasted_iota(jnp.int32, (_BR, 1), 0)
        msk = rid < N
        v = h_ref[...]
        vs = jnp.where(msk, v, 0.0)
        vm = jnp.where(msk, v, -jnp.inf)
        psum = jnp.sum(vs, axis=0, keepdims=True)
        pmax = jnp.max(vm, axis=0, keepdims=True)

        @pl.when(i == 0)
        def _():
            o_ref[...] = jnp.zeros_like(o_ref)

        o_ref[0:1, :] += psum
        o_ref[1:2, :] = jnp.maximum(o_ref[1:2, :], pmax)

        @pl.when(i == _GRID - 1)
        def _():
            o_ref[0:1, :] = o_ref[0:1, :] * (1.0 / N)

    return pl.pallas_call(
        body,
        grid=(_GRID,),
        in_specs=[pl.BlockSpec((_BR, H), lambda i: (i, 0))],
        out_specs=pl.BlockSpec((8, H), lambda i: (0, 0)),
        out_shape=jax.ShapeDtypeStruct((8, H), jnp.float32),
    )(h)


def _tc_head(p1, p2, p3, pg2, pg3, w1, b1, w2, b2):
    def body(p1r, p2r, p3r, g2r, g3r, w1r, b1r, w2r, b2r, o_ref):
        sm = p1r[0:1] + p2r[0:1] + p3r[0:1]
        sx = p1r[1:2] + p2r[1:2] + p3r[1:2]
        pm = p1r[0:1] + 2.0 * g2r[0:1]
        px = p1r[1:2] + g2r[1:2] + g3r[1:2]
        merged = jnp.concatenate([sm, sx, pm, px], axis=1)
        f1 = jnp.maximum(
            jnp.dot(merged, w1r[...], preferred_element_type=jnp.float32)
            + b1r[...], 0.0)
        f2 = (jnp.dot(f1, w2r[...], preferred_element_type=jnp.float32)
              + b2r[...])
        o_ref[...] = jnp.broadcast_to(jax.nn.sigmoid(f2), (8, 128))

    specs = [pl.BlockSpec(p.shape, lambda i: (0, 0))
             for p in (p1, p2, p3, pg2, pg3, w1, b1, w2, b2)]
    return pl.pallas_call(
        body,
        grid=(1,),
        in_specs=specs,
        out_specs=pl.BlockSpec((8, 128), lambda i: (0, 0)),
        out_shape=jax.ShapeDtypeStruct((8, 128), jnp.float32),
    )(p1, p2, p3, pg2, pg3, w1, b1, w2, b2)


def _unpanel(aggP, panels, nch, npc):
    a = aggP.reshape(panels, nch, npc, 128)
    return jnp.moveaxis(a, 0, 2).reshape(nch * npc, panels * 128)


def kernel(x, edge_index, W1, b1, W2, b2, W3, b3, Wg1, al1, ar1, bg1, Wg2,
           al2, ar2, bg2, Wg3, al3, ar3, bg3, fc1_W, fc1_b, fc2_W, fc2_b):
    src = edge_index[0]
    dst = edge_index[1]
    order = jnp.argsort(dst)
    dst_s = dst[order]
    src_s = src[order]
    srcp2, dstl2, bounds2 = _build_layout(src_s, dst_s, NCH2, NPC2, K2)
    srcp8, dstl8, bounds8 = _build_layout(src_s, dst_s, NCH8, NPC8, K8)
    arange_n = jnp.arange(N, dtype=jnp.int32)
    lo = jnp.searchsorted(dst_s, arange_n)
    hi = jnp.searchsorted(dst_s, arange_n + 1)
    deg_in = (hi - lo).astype(jnp.float32)
    srt = jnp.sort(src)
    lo2 = jnp.searchsorted(srt, arange_n)
    hi2 = jnp.searchsorted(srt, arange_n + 1)
    deg_out = (hi2 - lo2).astype(jnp.float32)
    degout_r = jnp.broadcast_to(
        jnp.pad(deg_out, (0, N_PAD - N))[:, None], (N_PAD, 128))
    degin_r = jnp.broadcast_to(
        jnp.pad(deg_in, (0, N_PAD - N))[:, None], (N_PAD, 128))

    x_pad = jnp.pad(x, ((0, N_PAD - N), (0, 0)))

    def gcn_layer(hin, W, b, panels, segk):
        t = _tc_scale(hin, degout_r)
        tP = t.reshape(N_PAD * panels, 128)
        aggP = segk(tP, srcp2, dstl2, bounds2)
        agg = _unpanel(aggP, panels, NCH2, NPC2)
        return _tc_mm(agg, W, degin_r, b.reshape(1, -1))

    gcn1 = gcn_layer(x_pad, W1, b1, 1, _sc_gcn1)
    gcn2 = gcn_layer(gcn1, W2, b2, 2, _sc_gcn2)
    gcn3 = gcn_layer(gcn2, W3, b3, 2, _sc_gcn2)

    def gat_layer(hin, Wg, al, ar, bg):
        alp = jnp.pad(al, ((0, 8 - HEADS), (0, 0)))
        arp = jnp.pad(ar, ((0, 8 - HEADS), (0, 0)))
        z, elp, erp = _tc_mm_z(hin, Wg, alp, arp)
        elp8 = jnp.pad(elp, ((0, 8), (0, 0)))
        erp8 = jnp.pad(erp, ((0, 8), (0, 0)))
        ee, s16 = _sc_alpha(elp8, erp8, srcp8, dstl8, bounds8)
        z8 = z.reshape(N_PAD * _NP8, 128)
        aggP = _sc_gat(z8, srcp8, dstl8, ee.reshape(-1), s16.reshape(-1),
                       bounds8)
        agg = _unpanel(aggP, _NP8, NCH8, NPC8)
        return _tc_gatpost(agg, bg.reshape(1, -1))

    gat1 = gat_layer(x_pad, Wg1, al1, ar1, bg1)
    gat2 = gat_layer(gat1, Wg2, al2, ar2, bg2)
    gat3 = gat_layer(gat2, Wg3, al3, ar3, bg3)

    p1 = _tc_pool(gcn1)
    p2 = _tc_pool(gcn2)
    p3 = _tc_pool(gcn3)
    pg2 = _tc_pool(gat2)
    pg3 = _tc_pool(gat3)
    w2p = jnp.pad(fc2_W, ((0, 0), (0, 128 - fc2_W.shape[1])))
    b2p = jnp.pad(fc2_b, (0, 128 - fc2_b.shape[0])).reshape(1, 128)
    out = _tc_head(p1, p2, p3, pg2, pg3, fc1_W, fc1_b.reshape(1, -1),
                   w2p, b2p)
    return out[0:1, 0:2]


# speedup vs baseline: 5.4898x; 1.1552x over previous
import functools
import jax
import jax.numpy as jnp
from jax import lax
from jax.experimental import pallas as pl
from jax.experimental.pallas import tpu as pltpu, tpu_sc as plsc

N = 10000
N_PAD = 10240
E = 320000
D_IN = 128
H = 256
HEADS = 4

K2 = 128
NCH2 = 20
NPC2 = N_PAD // NCH2
BUF2 = NPC2 + 128
ALIGN2 = 16 * K2
EPAD2 = E + NCH2 * ALIGN2

K8 = 64
NCH8 = 40
NPC8 = N_PAD // NCH8
BUF8 = NPC8 + 128
ALIGN8 = 16 * K8
EPAD8 = E + NCH8 * ALIGN8

_mesh = plsc.VectorSubcoreMesh(core_axis_name="c", subcore_axis_name="s")


def _build_layout(src_s, dst_s, nch, npc, k):
    align = 16 * k
    epad = E + nch * align
    cid = dst_s // npc
    cstart = jnp.searchsorted(dst_s, jnp.arange(nch, dtype=jnp.int32) * npc)
    cstart = cstart.astype(jnp.int32)
    cnt = jnp.diff(jnp.concatenate([cstart, jnp.array([E], jnp.int32)]))
    cntp = ((cnt + align - 1) // align) * align
    P = jnp.concatenate([jnp.zeros((1,), jnp.int32),
                         jnp.cumsum(cntp)[:-1].astype(jnp.int32)])
    pos = P[cid] + (jnp.arange(E, dtype=jnp.int32) - cstart[cid])
    srcp = jnp.zeros((epad,), jnp.int32).at[pos].set(src_s)
    dstl = jnp.full((epad,), npc, jnp.int32).at[pos].set(
        dst_s - cid.astype(jnp.int32) * npc)
    q = (cntp // 16).astype(jnp.int32)
    idx = jnp.arange(nch, dtype=jnp.int32) * 16
    bounds = (jnp.zeros((nch * 16,), jnp.int32)
              .at[idx].set(P).at[idx + 1].set(q))
    return srcp, dstl, bounds


def _sc_gcn(panels):
    scratch = [pltpu.VMEM((NCH2 * 16,), jnp.int32)]
    scratch += [pltpu.VMEM((K2,), jnp.int32)]
    scratch += [pltpu.VMEM((K2,), jnp.int32) for _ in range(panels)]
    scratch += [pltpu.VMEM((K2,), jnp.int32) for _ in range(panels)]
    scratch += [pltpu.VMEM((K2, 128), jnp.float32) for _ in range(panels)]
    scratch += [pltpu.VMEM((64, 128), jnp.float32)]
    scratch += [pltpu.VMEM_SHARED((BUF2, 128), jnp.float32)
                for _ in range(panels)]
    scratch += [pltpu.SemaphoreType.DMA for _ in range(panels)]

    @functools.partial(
        pl.kernel,
        out_type=jax.ShapeDtypeStruct((panels * NCH2 * NPC2, 128),
                                      jnp.float32),
        mesh=_mesh,
        scratch_types=scratch,
    )
    def k(featP, srcp, dstl, bounds, out, bounds_v, sidx2, *refs):
        pidx = refs[:panels]
        didx = refs[panels:2 * panels]
        rows = refs[2 * panels:3 * panels]
        zblk = refs[3 * panels]
        sbuf = refs[3 * panels + 1:3 * panels + 1 + panels]
        sems = refs[3 * panels + 1 + panels:]
        c = lax.axis_index("c")
        s = lax.axis_index("s")
        for i in range(64):
            for j in range(8):
                zblk[i, pl.ds(j * 16, 16)] = jnp.zeros((16,), jnp.float32)
        pltpu.sync_copy(bounds, bounds_v)

        def chunk_body(cc, chunk_carry):
            cg = c * (NCH2 // 2) + cc
            for p in range(panels):
                pltpu.sync_copy(
                    zblk.at[pl.ds(0, 40), :],
                    sbuf[p].at[pl.ds(pl.multiple_of(s * 40, 8), 40), :])
            plsc.subcore_barrier()
            bv = bounds_v[pl.ds(pl.multiple_of(cg * 16, 16), 16)]
            P = bv[0]
            q = bv[1]
            start = P + s * q
            nb = q // K2

            def body(i, carry):
                off = pl.multiple_of(start + i * K2, 8)
                pltpu.sync_copy(srcp.at[pl.ds(off, K2)], sidx2)
                handles = []
                for p in range(panels):
                    pltpu.sync_copy(dstl.at[pl.ds(off, K2)], didx[p])
                    if panels == 1:
                        pltpu.sync_copy(srcp.at[pl.ds(off, K2)], pidx[p])
                    else:
                        for j in range(K2 // 16):
                            sl = pl.ds(j * 16, 16)
                            pidx[p][sl] = sidx2[sl] * panels + p
                    handles.append(pltpu.async_copy(featP.at[pidx[p]],
                                                    rows[p], sems[p]))
                for h in handles:
                    h.wait()
                shandles = [
                    pltpu.async_copy(rows[p], sbuf[p].at[didx[p]], sems[p],
                                     add=True)
                    for p in range(panels)]
                for h in shandles:
                    h.wait()
                return carry

            lax.fori_loop(0, nb, body, 0)
            plsc.subcore_barrier()
            for p in range(panels):
                pltpu.sync_copy(
                    sbuf[p].at[pl.ds(pl.multiple_of(s * 32, 8), 32), :],
                    out.at[pl.ds(pl.multiple_of(
                        (p * NCH2 + cg) * NPC2 + s * 32, 8), 32), :])
            plsc.subcore_barrier()
            return chunk_carry

        lax.fori_loop(0, NCH2 // 2, chunk_body, 0)

    return k


_sc_gcn1 = _sc_gcn(1)
_sc_gcn2 = _sc_gcn(2)


@functools.partial(
    pl.kernel,
    out_type=[jax.ShapeDtypeStruct((EPAD8, 16), jnp.float32),
              jax.ShapeDtypeStruct((NCH8 * BUF8, 16), jnp.float32)],
    mesh=_mesh,
    scratch_types=[
        pltpu.VMEM((NCH8 * 16,), jnp.int32),
        pltpu.VMEM((K8,), jnp.int32),
        pltpu.VMEM((K8,), jnp.int32),
        pltpu.VMEM((K8,), jnp.int32),
        pltpu.VMEM((K8,), jnp.int32),
        pltpu.VMEM((K8, 128), jnp.float32),
        pltpu.VMEM((K8, 128), jnp.float32),
        pltpu.VMEM((K8, 128), jnp.float32),
        pltpu.VMEM((K8, 16), jnp.float32),
        pltpu.VMEM((64, 128), jnp.float32),
        pltpu.VMEM((24, 128), jnp.float32),
        pltpu.VMEM((24, 16), jnp.float32),
        pltpu.VMEM_SHARED((BUF8, 128), jnp.float32),
        pltpu.SemaphoreType.DMA,
        pltpu.SemaphoreType.DMA,
    ],
)
def _sc_alpha(elT, erT, srcp, dstl, bounds, ee_out, s_out, bounds_v,
              sidx, didx, didx2, didxg, elb, erb, eeb, ee16, zblk,
              sstage, s16b, s2, sem, sem2):
    c = lax.axis_index("c")
    s = lax.axis_index("s")
    for i in range(64):
        for j in range(8):
            zblk[i, pl.ds(j * 16, 16)] = jnp.zeros((16,), jnp.float32)
    pltpu.sync_copy(bounds, bounds_v)

    def chunk_body(cc, chunk_carry):
        cg = c * (NCH8 // 2) + cc
        base = cg * NPC8
        pltpu.sync_copy(zblk.at[pl.ds(0, 24), :],
                        s2.at[pl.ds(pl.multiple_of(s * 24, 8), 24), :])
        plsc.subcore_barrier()
        bv = bounds_v[pl.ds(pl.multiple_of(cg * 16, 16), 16)]
        P = bv[0]
        q = bv[1]
        start = P + s * q
        nb = q // K8

        def body(i, carry):
            off = pl.multiple_of(start + i * K8, 8)
            pltpu.sync_copy(srcp.at[pl.ds(off, K8)], sidx)
            pltpu.sync_copy(dstl.at[pl.ds(off, K8)], didx)
            pltpu.sync_copy(dstl.at[pl.ds(off, K8)], didx2)
            for j in range(K8 // 16):
                sl = pl.ds(j * 16, 16)
                didxg[sl] = didx2[sl] + base
            h1 = pltpu.async_copy(elT.at[sidx], elb, sem)
            h2 = pltpu.async_copy(erT.at[didxg], erb, sem2)
            h1.wait()
            h2.wait()
            for e in range(K8):
                v = elb[e, pl.ds(0, 16)] + erb[e, pl.ds(0, 16)]
                v = jnp.where(v > 0, v, 0.2 * v)
                v = jnp.exp(v)
                v = jnp.where(lax.iota(jnp.int32, 16) < 4, v, 0.0)
                eeb[e, pl.ds(0, 16)] = v
                ee16[e, pl.ds(0, 16)] = v
                for j in range(1, 8):
                    eeb[e, pl.ds(j * 16, 16)] = jnp.zeros((16,), jnp.float32)
            pltpu.sync_copy(eeb, s2.at[didx], add=True)
            pltpu.sync_copy(ee16, ee_out.at[pl.ds(off, K8), :])
            return carry

        lax.fori_loop(0, nb, body, 0)
        plsc.subcore_barrier()
        pltpu.sync_copy(s2.at[pl.ds(pl.multiple_of(s * 24, 8), 24), :], sstage)
        for r in range(24):
            s16b[r, pl.ds(0, 16)] = sstage[r, pl.ds(0, 16)]
        pltpu.sync_copy(
            s16b,
            s_out.at[pl.ds(pl.multiple_of(cg * BUF8 + s * 24, 8), 24), :])
        plsc.subcore_barrier()
        return chunk_carry

    lax.fori_loop(0, NCH8 // 2, chunk_body, 0)
    return


_NP8 = 8

_NPAIR = 4

_gat_scratch = [pltpu.VMEM((NCH8 * 16,), jnp.int32)]
_gat_scratch += [pltpu.VMEM((K8,), jnp.int32)]
_gat_scratch += [pltpu.VMEM((K8,), jnp.int32)]
_gat_scratch += [pltpu.VMEM((2 * K8,), jnp.int32) for _ in range(_NPAIR)]
_gat_scratch += [pltpu.VMEM((2 * K8,), jnp.int32) for _ in range(_NPAIR)]
_gat_scratch += [pltpu.VMEM((2 * K8, 128), jnp.float32) for _ in range(_NPAIR)]
_gat_scratch += [pltpu.VMEM((K8 * 16,), jnp.float32)]
_gat_scratch += [pltpu.VMEM((BUF8 * 16,), jnp.float32)]
_gat_scratch += [pltpu.VMEM((64, 128), jnp.float32)]
_gat_scratch += [pltpu.VMEM_SHARED((_NP8 * BUF8, 128), jnp.float32)]
_gat_scratch += [pltpu.SemaphoreType.DMA for _ in range(_NPAIR)]


@functools.partial(
    pl.kernel,
    out_type=jax.ShapeDtypeStruct((_NP8 * NCH8 * NPC8, 128), jnp.float32),
    mesh=_mesh,
    scratch_types=_gat_scratch,
)
def _sc_gat(z8, srcp, dstl, ee_flat, s_flat, bounds, out, bounds_v,
            sidx2, didx2, *refs):
    pidx = refs[:_NPAIR]
    didxs = refs[_NPAIR:2 * _NPAIR]
    zp = refs[2 * _NPAIR:3 * _NPAIR]
    eebuf = refs[3 * _NPAIR]
    svbuf = refs[3 * _NPAIR + 1]
    zblk = refs[3 * _NPAIR + 2]
    sbuf = refs[3 * _NPAIR + 3]
    sems = refs[3 * _NPAIR + 4:]
    c = lax.axis_index("c")
    s = lax.axis_index("s")
    for i in range(64):
        for j in range(8):
            zblk[i, pl.ds(j * 16, 16)] = jnp.zeros((16,), jnp.float32)
    pltpu.sync_copy(bounds, bounds_v)

    def chunk_body(cc, chunk_carry):
        cg = c * (NCH8 // 2) + cc
        for kk in range(3):
            pltpu.sync_copy(
                zblk,
                sbuf.at[pl.ds(pl.multiple_of(s * 192 + kk * 64, 8), 64), :])
        pltpu.sync_copy(
            s_flat.at[pl.ds(pl.multiple_of(cg * BUF8 * 16, 8), BUF8 * 16)],
            svbuf)
        plsc.subcore_barrier()
        bv = bounds_v[pl.ds(pl.multiple_of(cg * 16, 16), 16)]
        P = bv[0]
        q = bv[1]
        start = P + s * q
        nb = q // K8

        def body(i, carry):
            off = pl.multiple_of(start + i * K8, 8)
            pltpu.sync_copy(srcp.at[pl.ds(off, K8)], sidx2)
            pltpu.sync_copy(dstl.at[pl.ds(off, K8)], didx2)
            pltpu.sync_copy(
                ee_flat.at[pl.ds(pl.multiple_of(off * 16, 8), K8 * 16)],
                eebuf)
            handles = []
            for pp in range(_NPAIR):
                for j in range(K8 // 16):
                    sl = pl.ds(j * 16, 16)
                    sl2 = pl.ds(K8 + j * 16, 16)
                    s_v = sidx2[sl]
                    d_v = didx2[sl]
                    pidx[pp][sl] = s_v * _NP8 + 2 * pp
                    pidx[pp][sl2] = s_v * _NP8 + 2 * pp + 1
                    didxs[pp][sl] = d_v + (2 * pp) * BUF8
                    didxs[pp][sl2] = d_v + (2 * pp + 1) * BUF8
                handles.append(pltpu.async_copy(z8.at[pidx[pp]], zp[pp],
                                                sems[pp]))
            for h in handles:
                h.wait()

            def gbody(g, carry2):
                for t in range(16):
                    e = g * 16 + t
                    ev = eebuf[pl.ds(pl.multiple_of(e * 16, 16), 16)]
                    dv = didx2[pl.ds(pl.multiple_of(g * 16, 16), 16)]
                    d = dv[t]
                    sv = svbuf[pl.ds(pl.multiple_of(d * 16, 16), 16)]
                    av = ev / (sv + 1e-9)
                    a = (av[0], av[1], av[2], av[3])
                    for pp in range(_NPAIR):
                        bb = jnp.full((16,), a[pp], jnp.float32)
                        for j in range(8):
                            sl = pl.ds(j * 16, 16)
                            zp[pp][e, sl] = zp[pp][e, sl] * bb
                            zp[pp][K8 + e, sl] = zp[pp][K8 + e, sl] * bb
                return carry2

            lax.fori_loop(0, K8 // 16, gbody, 0)
            shandles = [
                pltpu.async_copy(zp[pp], sbuf.at[didxs[pp]], sems[pp],
                                 add=True)
                for pp in range(_NPAIR)]
            for h in shandles:
                h.wait()
            return carry

        lax.fori_loop(0, nb, body, 0)
        plsc.subcore_barrier()
        for p in range(_NP8):
            pltpu.sync_copy(
                sbuf.at[pl.ds(pl.multiple_of(p * BUF8 + s * 16, 8), 16), :],
                out.at[pl.ds(pl.multiple_of(
                    (p * NCH8 + cg) * NPC8 + s * 16, 8), 16), :])
        plsc.subcore_barrier()
        return chunk_carry

    lax.fori_loop(0, NCH8 // 2, chunk_body, 0)
    return


_BR = 1024
_GRID = N_PAD // _BR


def _norm_from_deg(dv):
    return jnp.where(dv > 0, lax.rsqrt(jnp.maximum(dv, 1.0)), 0.0)


def _tc_scale(a, deg_r):
    f = a.shape[1]

    def body(a_ref, d_ref, o_ref):
        ns = _norm_from_deg(d_ref[...])
        if f > 128:
            ns = jnp.concatenate([ns] * (f // 128), axis=1)
        o_ref[...] = a_ref[...] * ns

    return pl.pallas_call(
        body,
        grid=(_GRID,),
        in_specs=[pl.BlockSpec((_BR, f), lambda i: (i, 0)),
                  pl.BlockSpec((_BR, 128), lambda i: (i, 0))],
        out_specs=pl.BlockSpec((_BR, f), lambda i: (i, 0)),
        out_shape=jax.ShapeDtypeStruct((N_PAD, f), jnp.float32),
    )(a, deg_r)


def _tc_mm(a, w, deg_r, b):
    kd, m = w.shape

    def body(a_ref, w_ref, d_ref, b_ref, o_ref):
        acc = jnp.dot(a_ref[...], w_ref[...],
                      preferred_element_type=jnp.float32)
        nd = _norm_from_deg(d_ref[...])
        if m > 128:
            nd = jnp.concatenate([nd] * (m // 128), axis=1)
        o_ref[...] = jnp.maximum(acc * nd + b_ref[...], 0.0)

    return pl.pallas_call(
        body,
        grid=(_GRID,),
        in_specs=[pl.BlockSpec((_BR, kd), lambda i: (i, 0)),
                  pl.BlockSpec((kd, m), lambda i: (0, 0)),
                  pl.BlockSpec((_BR, 128), lambda i: (i, 0)),
                  pl.BlockSpec((1, m), lambda i: (0, 0))],
        out_specs=pl.BlockSpec((_BR, m), lambda i: (i, 0)),
        out_shape=jax.ShapeDtypeStruct((N_PAD, m), jnp.float32),
    )(a, w, deg_r, b)


def _tc_mm_z(a, w, alp, arp):
    kd, m = w.shape

    def body(a_ref, w_ref, al_ref, ar_ref, z_ref, el_ref, er_ref):
        z = jnp.dot(a_ref[...], w_ref[...],
                    preferred_element_type=jnp.float32)
        z_ref[...] = z
        z3 = z.reshape(_BR, HEADS, H)
        al = al_ref[...][0:HEADS]
        ar = ar_ref[...][0:HEADS]
        el = jnp.sum(z3 * al[None], axis=-1)
        er = jnp.sum(z3 * ar[None], axis=-1)
        el_ref[...] = jnp.pad(el, ((0, 0), (0, 128 - HEADS)))
        er_ref[...] = jnp.pad(er, ((0, 0), (0, 128 - HEADS)))

    return pl.pallas_call(
        body,
        grid=(_GRID,),
        in_specs=[pl.BlockSpec((_BR, kd), lambda i: (i, 0)),
                  pl.BlockSpec((kd, m), lambda i: (0, 0)),
                  pl.BlockSpec((8, H), lambda i: (0, 0)),
                  pl.BlockSpec((8, H), lambda i: (0, 0))],
        out_specs=[pl.BlockSpec((_BR, m), lambda i: (i, 0)),
                   pl.BlockSpec((_BR, 128), lambda i: (i, 0)),
                   pl.BlockSpec((_BR, 128), lambda i: (i, 0))],
        out_shape=[jax.ShapeDtypeStruct((N_PAD, m), jnp.float32),
                   jax.ShapeDtypeStruct((N_PAD, 128), jnp.float32),
                   jax.ShapeDtypeStruct((N_PAD, 128), jnp.float32)],
    )(a, w, alp, arp)


def _tc_gatpost(agg, bg):
    def body(a_ref, b_ref, o_ref):
        v = jnp.maximum(a_ref[...] + b_ref[...], 0.0)
        o_ref[...] = jnp.mean(v.reshape(_BR, HEADS, H), axis=1)

    return pl.pallas_call(
        body,
        grid=(_GRID,),
        in_specs=[pl.BlockSpec((_BR, HEADS * H), lambda i: (i, 0)),
                  pl.BlockSpec((1, HEADS * H), lambda i: (0, 0))],
        out_specs=pl.BlockSpec((_BR, H), lambda i: (i, 0)),
        out_shape=jax.ShapeDtypeStruct((N_PAD, H), jnp.float32),
    )(agg, bg)


def _tc_pool(h):
    def body(h_ref, o_ref):
        i = pl.program_id(0)
        rid = i * _BR + lax.broadcasted_iota(jnp.int32, (_BR, 1), 0)
        msk = rid < N
        v = h_ref[...]
        vs = jnp.where(msk, v, 0.0)
        vm = jnp.where(msk, v, -jnp.inf)
        psum = jnp.sum(vs, axis=0, keepdims=True)
        pmax = jnp.max(vm, axis=0, keepdims=True)

        @pl.when(i == 0)
        def _():
            o_ref[...] = jnp.zeros_like(o_ref)

        o_ref[0:1, :] += psum
        o_ref[1:2, :] = jnp.maximum(o_ref[1:2, :], pmax)

        @pl.when(i == _GRID - 1)
        def _():
            o_ref[0:1, :] = o_ref[0:1, :] * (1.0 / N)

    return pl.pallas_call(
        body,
        grid=(_GRID,),
        in_specs=[pl.BlockSpec((_BR, H), lambda i: (i, 0))],
        out_specs=pl.BlockSpec((8, H), lambda i: (0, 0)),
        out_shape=jax.ShapeDtypeStruct((8, H), jnp.float32),
    )(h)


def _tc_head(p1, p2, p3, pg2, pg3, w1, b1, w2, b2):
    def body(p1r, p2r, p3r, g2r, g3r, w1r, b1r, w2r, b2r, o_ref):
        sm = p1r[0:1] + p2r[0:1] + p3r[0:1]
        sx = p1r[1:2] + p2r[1:2] + p3r[1:2]
        pm = p1r[0:1] + 2.0 * g2r[0:1]
        px = p1r[1:2] + g2r[1:2] + g3r[1:2]
        merged = jnp.concatenate([sm, sx, pm, px], axis=1)
        f1 = jnp.maximum(
            jnp.dot(merged, w1r[...], preferred_element_type=jnp.float32)
            + b1r[...], 0.0)
        f2 = (jnp.dot(f1, w2r[...], preferred_element_type=jnp.float32)
              + b2r[...])
        o_ref[...] = jnp.broadcast_to(jax.nn.sigmoid(f2), (8, 128))

    specs = [pl.BlockSpec(p.shape, lambda i: (0, 0))
             for p in (p1, p2, p3, pg2, pg3, w1, b1, w2, b2)]
    return pl.pallas_call(
        body,
        grid=(1,),
        in_specs=specs,
        out_specs=pl.BlockSpec((8, 128), lambda i: (0, 0)),
        out_shape=jax.ShapeDtypeStruct((8, 128), jnp.float32),
    )(p1, p2, p3, pg2, pg3, w1, b1, w2, b2)


def _unpanel(aggP, panels, nch, npc):
    a = aggP.reshape(panels, nch, npc, 128)
    return jnp.moveaxis(a, 0, 2).reshape(nch * npc, panels * 128)


def kernel(x, edge_index, W1, b1, W2, b2, W3, b3, Wg1, al1, ar1, bg1, Wg2,
           al2, ar2, bg2, Wg3, al3, ar3, bg3, fc1_W, fc1_b, fc2_W, fc2_b):
    src = edge_index[0]
    dst = edge_index[1]
    order = jnp.argsort(dst)
    dst_s = dst[order]
    src_s = src[order]
    srcp2, dstl2, bounds2 = _build_layout(src_s, dst_s, NCH2, NPC2, K2)
    srcp8, dstl8, bounds8 = _build_layout(src_s, dst_s, NCH8, NPC8, K8)
    arange_n = jnp.arange(N, dtype=jnp.int32)
    lo = jnp.searchsorted(dst_s, arange_n)
    hi = jnp.searchsorted(dst_s, arange_n + 1)
    deg_in = (hi - lo).astype(jnp.float32)
    srt = jnp.sort(src)
    lo2 = jnp.searchsorted(srt, arange_n)
    hi2 = jnp.searchsorted(srt, arange_n + 1)
    deg_out = (hi2 - lo2).astype(jnp.float32)
    degout_r = jnp.broadcast_to(
        jnp.pad(deg_out, (0, N_PAD - N))[:, None], (N_PAD, 128))
    degin_r = jnp.broadcast_to(
        jnp.pad(deg_in, (0, N_PAD - N))[:, None], (N_PAD, 128))

    x_pad = jnp.pad(x, ((0, N_PAD - N), (0, 0)))

    def gcn_layer(hin, W, b, panels, segk):
        t = _tc_scale(hin, degout_r)
        tP = t.reshape(N_PAD * panels, 128)
        aggP = segk(tP, srcp2, dstl2, bounds2)
        agg = _unpanel(aggP, panels, NCH2, NPC2)
        return _tc_mm(agg, W, degin_r, b.reshape(1, -1))

    gcn1 = gcn_layer(x_pad, W1, b1, 1, _sc_gcn1)
    gcn2 = gcn_layer(gcn1, W2, b2, 2, _sc_gcn2)
    gcn3 = gcn_layer(gcn2, W3, b3, 2, _sc_gcn2)

    def gat_layer(hin, Wg, al, ar, bg):
        alp = jnp.pad(al, ((0, 8 - HEADS), (0, 0)))
        arp = jnp.pad(ar, ((0, 8 - HEADS), (0, 0)))
        z, elp, erp = _tc_mm_z(hin, Wg, alp, arp)
        elp8 = jnp.pad(elp, ((0, 8), (0, 0)))
        erp8 = jnp.pad(erp, ((0, 8), (0, 0)))
        ee, s16 = _sc_alpha(elp8, erp8, srcp8, dstl8, bounds8)
        z8 = z.reshape(N_PAD * _NP8, 128)
        aggP = _sc_gat(z8, srcp8, dstl8, ee.reshape(-1), s16.reshape(-1),
                       bounds8)
        agg = _unpanel(aggP, _NP8, NCH8, NPC8)
        return _tc_gatpost(agg, bg.reshape(1, -1))

    gat1 = gat_layer(x_pad, Wg1, al1, ar1, bg1)
    gat2 = gat_layer(gat1, Wg2, al2, ar2, bg2)
    gat3 = gat_layer(gat2, Wg3, al3, ar3, bg3)

    p1 = _tc_pool(gcn1)
    p2 = _tc_pool(gcn2)
    p3 = _tc_pool(gcn3)
    pg2 = _tc_pool(gat2)
    pg3 = _tc_pool(gat3)
    w2p = jnp.pad(fc2_W, ((0, 0), (0, 128 - fc2_W.shape[1])))
    b2p = jnp.pad(fc2_b, (0, 128 - fc2_b.shape[0])).reshape(1, 128)
    out = _tc_head(p1, p2, p3, pg2, pg3, fc1_W, fc1_b.reshape(1, -1),
                   w2p, b2p)
    return out[0:1, 0:2]
